# baseline (speedup 1.0000x reference)
import functools

import jax
import jax.numpy as jnp
from jax import lax
from jax.experimental import pallas as pl
from jax.experimental.pallas import tpu as pltpu
from jax.experimental.pallas import tpu_sc as plsc

B = 4096
D = 64
L = 16
NC = 2
NS = 16
NW = NC * NS
BPW = B // NW
NG = BPW // L

RB = 512


def _sc_scores(user, pos_item, neg_item, user_table, item_table):
    mesh = plsc.VectorSubcoreMesh(core_axis_name="c", subcore_axis_name="s")

    @functools.partial(
        pl.kernel,
        mesh=mesh,
        compiler_params=pltpu.CompilerParams(
            needs_layout_passes=False, use_tc_tiling_on_sc=False),
        out_type=[
            jax.ShapeDtypeStruct((B,), jnp.float32),
            jax.ShapeDtypeStruct((B,), jnp.float32),
        ],
        scratch_types=[
            pltpu.VMEM((BPW,), jnp.int32),
            pltpu.VMEM((BPW,), jnp.int32),
            pltpu.VMEM((BPW,), jnp.int32),
            pltpu.VMEM((BPW, D), jnp.float32),
            pltpu.VMEM((BPW, D), jnp.float32),
            pltpu.VMEM((BPW, D), jnp.float32),
            pltpu.VMEM((BPW,), jnp.float32),
            pltpu.VMEM((BPW,), jnp.float32),
            pltpu.SemaphoreType.DMA,
        ],
    )
    def k(u_hbm, p_hbm, n_hbm, ut_hbm, it_hbm, ps_out, ns_out,
          uidx, pidx, nidx, urows, prows, nrows, ps_v, ns_v, sem):
        wid = lax.axis_index("s") * NC + lax.axis_index("c")
        base = wid * BPW
        pltpu.sync_copy(u_hbm.at[pl.ds(base, BPW)], uidx)
        pltpu.sync_copy(p_hbm.at[pl.ds(base, BPW)], pidx)
        pltpu.sync_copy(n_hbm.at[pl.ds(base, BPW)], nidx)
        cu = pltpu.async_copy(ut_hbm.at[uidx], urows, sem)
        cp = pltpu.async_copy(it_hbm.at[pidx], prows, sem)
        cn = pltpu.async_copy(it_hbm.at[nidx], nrows, sem)
        cu.wait()
        cp.wait()
        cn.wait()

        lane = lax.iota(jnp.int32, L)

        def group(g, carry):
            pvec = jnp.zeros((L,), jnp.float32)
            nvec = jnp.zeros((L,), jnp.float32)
            for j in range(L):
                r = g * L + j
                pacc = jnp.zeros((L,), jnp.float32)
                nacc = jnp.zeros((L,), jnp.float32)
                for c in range(D // L):
                    sl = pl.ds(c * L, L)
                    uv = urows[r, sl]
                    pacc = pacc + uv * prows[r, sl]
                    nacc = nacc + uv * nrows[r, sl]
                mask = lane == j
                pvec = jnp.where(mask, jnp.sum(pacc), pvec)
                nvec = jnp.where(mask, jnp.sum(nacc), nvec)
            ps_v[pl.ds(g * L, L)] = pvec
            ns_v[pl.ds(g * L, L)] = nvec
            return carry

        lax.fori_loop(0, NG, group, 0)
        pltpu.sync_copy(ps_v, ps_out.at[pl.ds(base, BPW)])
        pltpu.sync_copy(ns_v, ns_out.at[pl.ds(base, BPW)])

    return k(user, pos_item, neg_item, user_table, item_table)


def _tc_loss_sum(p, n):
    p2 = p.reshape(B, 1)
    n2 = n.reshape(1, B)

    def body(p_ref, n_ref, out_ref):
        i = pl.program_id(0)
        z = n_ref[...] - p_ref[...]
        m = jnp.maximum(z, 0.0)
        t = 1.0 + jnp.exp(-jnp.abs(z))
        acc = t[:, 0:128]
        for c in range(1, B // 128):
            acc = acc * t[:, c * 128:(c + 1) * 128]
        part = jnp.sum(m) + jnp.sum(jnp.log(acc))

        @pl.when(i == 0)
        def _():
            out_ref[...] = jnp.zeros_like(out_ref)

        out_ref[...] += jnp.reshape(part, (1, 1))

    out = pl.pallas_call(
        body,
        grid=(B // RB,),
        in_specs=[
            pl.BlockSpec((RB, 1), lambda i: (i, 0)),
            pl.BlockSpec((1, B), lambda i: (0, 0)),
        ],
        out_specs=pl.BlockSpec((1, 1), lambda i: (0, 0)),
        out_shape=jax.ShapeDtypeStruct((1, 1), jnp.float32),
    )(p2, n2)
    return out[0, 0]


def kernel(user, pos_item, neg_item, user_table, item_table):
    p, nvec = _sc_scores(user, pos_item, jnp.reshape(neg_item, (-1,)),
                         user_table, item_table)
    loss = _tc_loss_sum(p, nvec) / (B * B)
    logits = p.reshape(B, 1)
    return (logits, loss)


# speedup vs baseline: 1.5442x; 1.5442x over previous
import functools

import jax
import jax.numpy as jnp
from jax import lax
from jax.experimental import pallas as pl
from jax.experimental.pallas import tpu as pltpu
from jax.experimental.pallas import tpu_sc as plsc

B = 4096
D = 64
L = 16
NC = 2
NS = 16
NW = NC * NS
BPW = B // NW
NG = BPW // L

RB = 512


def _sc_scores(user, pos_item, neg_item, user_table, item_table):
    mesh = plsc.VectorSubcoreMesh(core_axis_name="c", subcore_axis_name="s")

    @functools.partial(
        pl.kernel,
        mesh=mesh,
        compiler_params=pltpu.CompilerParams(needs_layout_passes=False),
        out_type=[
            jax.ShapeDtypeStruct((B,), jnp.float32),
            jax.ShapeDtypeStruct((B,), jnp.float32),
        ],
        scratch_types=[
            pltpu.VMEM((BPW,), jnp.int32),
            pltpu.VMEM((BPW,), jnp.int32),
            pltpu.VMEM((BPW,), jnp.int32),
            pltpu.VMEM((BPW, D), jnp.float32),
            pltpu.VMEM((BPW, D), jnp.float32),
            pltpu.VMEM((BPW, D), jnp.float32),
            pltpu.VMEM((BPW,), jnp.float32),
            pltpu.VMEM((BPW,), jnp.float32),
            pltpu.SemaphoreType.DMA,
        ],
    )
    def k(u_hbm, p_hbm, n_hbm, ut_hbm, it_hbm, ps_out, ns_out,
          uidx, pidx, nidx, urows, prows, nrows, ps_v, ns_v, sem):
        wid = lax.axis_index("s") * NC + lax.axis_index("c")
        base = wid * BPW
        pltpu.sync_copy(u_hbm.at[pl.ds(base, BPW)], uidx)
        pltpu.sync_copy(p_hbm.at[pl.ds(base, BPW)], pidx)
        pltpu.sync_copy(n_hbm.at[pl.ds(base, BPW)], nidx)

        CH = 16

        def fetch_chunk(c, carry):
            cbase = c * CH
            uvec = uidx[pl.ds(cbase, CH)]
            pvec = pidx[pl.ds(cbase, CH)]
            nvec = nidx[pl.ds(cbase, CH)]
            copies = []
            for j in range(CH):
                r = cbase + j
                copies.append(pltpu.async_copy(
                    ut_hbm.at[pl.ds(uvec[j], 1)], urows.at[pl.ds(r, 1)], sem))
                copies.append(pltpu.async_copy(
                    it_hbm.at[pl.ds(pvec[j], 1)], prows.at[pl.ds(r, 1)], sem))
                copies.append(pltpu.async_copy(
                    it_hbm.at[pl.ds(nvec[j], 1)], nrows.at[pl.ds(r, 1)], sem))
            for cpy in copies:
                cpy.wait()
            return carry

        lax.fori_loop(0, BPW // CH, fetch_chunk, 0)

        lane = lax.iota(jnp.int32, L)

        def group(g, carry):
            pvec = jnp.zeros((L,), jnp.float32)
            nvec = jnp.zeros((L,), jnp.float32)
            for j in range(L):
                r = g * L + j
                pacc = jnp.zeros((L,), jnp.float32)
                nacc = jnp.zeros((L,), jnp.float32)
                for c in range(D // L):
                    sl = pl.ds(c * L, L)
                    uv = urows[r, sl]
                    pacc = pacc + uv * prows[r, sl]
                    nacc = nacc + uv * nrows[r, sl]
                mask = lane == j
                pvec = jnp.where(mask, jnp.sum(pacc), pvec)
                nvec = jnp.where(mask, jnp.sum(nacc), nvec)
            ps_v[pl.ds(g * L, L)] = pvec
            ns_v[pl.ds(g * L, L)] = nvec
            return carry

        lax.fori_loop(0, NG, group, 0)
        pltpu.sync_copy(ps_v, ps_out.at[pl.ds(base, BPW)])
        pltpu.sync_copy(ns_v, ns_out.at[pl.ds(base, BPW)])

    return k(user, pos_item, neg_item, user_table, item_table)


def _tc_loss_sum(p, n):
    p2 = p.reshape(B, 1)
    n2 = n.reshape(1, B)

    def body(p_ref, n_ref, out_ref):
        i = pl.program_id(0)
        z = n_ref[...] - p_ref[...]
        m = jnp.maximum(z, 0.0)
        t = 1.0 + jnp.exp(-jnp.abs(z))
        acc = t[:, 0:128]
        for c in range(1, B // 128):
            acc = acc * t[:, c * 128:(c + 1) * 128]
        part = jnp.sum(m) + jnp.sum(jnp.log(acc))

        @pl.when(i == 0)
        def _():
            out_ref[...] = jnp.zeros_like(out_ref)

        out_ref[...] += jnp.reshape(part, (1, 1))

    out = pl.pallas_call(
        body,
        grid=(B // RB,),
        in_specs=[
            pl.BlockSpec((RB, 1), lambda i: (i, 0)),
            pl.BlockSpec((1, B), lambda i: (0, 0)),
        ],
        out_specs=pl.BlockSpec((1, 1), lambda i: (0, 0)),
        out_shape=jax.ShapeDtypeStruct((1, 1), jnp.float32),
    )(p2, n2)
    return out[0, 0]


def kernel(user, pos_item, neg_item, user_table, item_table):
    p, nvec = _sc_scores(user, pos_item, jnp.reshape(neg_item, (-1,)),
                         user_table, item_table)
    loss = _tc_loss_sum(p, nvec) / (B * B)
    logits = p.reshape(B, 1)
    return (logits, loss)


# speedup vs baseline: 3.9050x; 2.5288x over previous
import functools

import jax
import jax.numpy as jnp
from jax import lax
from jax.experimental import pallas as pl
from jax.experimental.pallas import tpu as pltpu
from jax.experimental.pallas import tpu_sc as plsc

B = 4096
D = 64
NROWS = 1000000
L = 16
NC = 2
NS = 16
NW = NC * NS
BPW = B // NW
NG = BPW // L
W = 128
WMAX = NROWS - W

RB = 512


def _sc_scores(user, pos_item, neg_item, user_table_t, item_table_t):
    mesh = plsc.VectorSubcoreMesh(core_axis_name="c", subcore_axis_name="s")

    @functools.partial(
        pl.kernel,
        mesh=mesh,
        compiler_params=pltpu.CompilerParams(needs_layout_passes=False),
        out_type=[
            jax.ShapeDtypeStruct((B,), jnp.float32),
            jax.ShapeDtypeStruct((B,), jnp.float32),
        ],
        scratch_types=[
            pltpu.VMEM((BPW,), jnp.int32),
            pltpu.VMEM((BPW,), jnp.int32),
            pltpu.VMEM((BPW,), jnp.int32),
            pltpu.VMEM((D, W), jnp.float32),
            pltpu.VMEM((D, W), jnp.float32),
            pltpu.VMEM((D, W), jnp.float32),
            pltpu.VMEM((BPW,), jnp.float32),
            pltpu.VMEM((BPW,), jnp.float32),
            pltpu.SemaphoreType.DMA,
        ],
    )
    def k(u_hbm, p_hbm, n_hbm, ut_hbm, it_hbm, ps_out, ns_out,
          uidx, pidx, nidx, wu, wp, wn, ps_v, ns_v, sem):
        wid = lax.axis_index("s") * NC + lax.axis_index("c")
        base = wid * BPW
        pltpu.sync_copy(u_hbm.at[pl.ds(base, BPW)], uidx)
        pltpu.sync_copy(p_hbm.at[pl.ds(base, BPW)], pidx)
        pltpu.sync_copy(n_hbm.at[pl.ds(base, BPW)], nidx)

        lane = lax.iota(jnp.int32, L)
        rows4 = [lax.iota(jnp.int32, L) + c * L for c in range(D // L)]

        def win_off(i):
            return i & jnp.int32(-W)

        def group(g, carry):
            gbase = g * L
            uvec = uidx[pl.ds(gbase, L)]
            pvec = pidx[pl.ds(gbase, L)]
            nvec = nidx[pl.ds(gbase, L)]
            psel = jnp.zeros((L,), jnp.float32)
            nsel = jnp.zeros((L,), jnp.float32)
            for j in range(L):
                iu, ip, iN = uvec[j], pvec[j], nvec[j]
                ou, op, oN = win_off(iu), win_off(ip), win_off(iN)
                cu = pltpu.async_copy(
                    ut_hbm.at[:, pl.ds(pl.multiple_of(ou, W), W)], wu, sem)
                cp = pltpu.async_copy(
                    it_hbm.at[:, pl.ds(pl.multiple_of(op, W), W)], wp, sem)
                cn = pltpu.async_copy(
                    it_hbm.at[:, pl.ds(pl.multiple_of(oN, W), W)], wn, sem)
                cu.wait()
                cp.wait()
                cn.wait()
                lu = jnp.full((L,), iu - ou, jnp.int32)
                lp = jnp.full((L,), ip - op, jnp.int32)
                ln = jnp.full((L,), iN - oN, jnp.int32)
                pacc = jnp.zeros((L,), jnp.float32)
                nacc = jnp.zeros((L,), jnp.float32)
                for c in range(D // L):
                    uv = plsc.load_gather(wu, [rows4[c], lu])
                    pv = plsc.load_gather(wp, [rows4[c], lp])
                    nv = plsc.load_gather(wn, [rows4[c], ln])
                    pacc = pacc + uv * pv
                    nacc = nacc + uv * nv
                mask = lane == j
                psel = jnp.where(mask, jnp.sum(pacc), psel)
                nsel = jnp.where(mask, jnp.sum(nacc), nsel)
            ps_v[pl.ds(gbase, L)] = psel
            ns_v[pl.ds(gbase, L)] = nsel
            return carry

        lax.fori_loop(0, NG, group, 0)
        pltpu.sync_copy(ps_v, ps_out.at[pl.ds(base, BPW)])
        pltpu.sync_copy(ns_v, ns_out.at[pl.ds(base, BPW)])

    return k(user, pos_item, neg_item, user_table_t, item_table_t)


def _tc_loss_sum(p, n):
    p2 = p.reshape(B, 1)
    n2 = n.reshape(1, B)

    def body(p_ref, n_ref, out_ref):
        i = pl.program_id(0)
        z = n_ref[...] - p_ref[...]
        m = jnp.maximum(z, 0.0)
        t = 1.0 + jnp.exp(-jnp.abs(z))
        acc = t[:, 0:128]
        for c in range(1, B // 128):
            acc = acc * t[:, c * 128:(c + 1) * 128]
        part = jnp.sum(m) + jnp.sum(jnp.log(acc))

        @pl.when(i == 0)
        def _():
            out_ref[...] = jnp.zeros_like(out_ref)

        out_ref[...] += jnp.reshape(part, (1, 1))

    out = pl.pallas_call(
        body,
        grid=(B // RB,),
        in_specs=[
            pl.BlockSpec((RB, 1), lambda i: (i, 0)),
            pl.BlockSpec((1, B), lambda i: (0, 0)),
        ],
        out_specs=pl.BlockSpec((1, 1), lambda i: (0, 0)),
        out_shape=jax.ShapeDtypeStruct((1, 1), jnp.float32),
    )(p2, n2)
    return out[0, 0]


def kernel(user, pos_item, neg_item, user_table, item_table):
    p, nvec = _sc_scores(user, pos_item, jnp.reshape(neg_item, (-1,)),
                         user_table.T, item_table.T)
    loss = _tc_loss_sum(p, nvec) / (B * B)
    logits = p.reshape(B, 1)
    return (logits, loss)


# speedup vs baseline: 4.7520x; 1.2169x over previous
import functools

import jax
import jax.numpy as jnp
from jax import lax
from jax.experimental import pallas as pl
from jax.experimental.pallas import tpu as pltpu
from jax.experimental.pallas import tpu_sc as plsc

B = 4096
D = 64
NROWS = 1000000
L = 16
NC = 2
NS = 16
NW = NC * NS
BPW = B // NW
NG = BPW // L
W = 128
WMAX = NROWS - W

RB = 512


def _sc_scores(user, pos_item, neg_item, user_table_t, item_table_t):
    mesh = plsc.VectorSubcoreMesh(core_axis_name="c", subcore_axis_name="s")

    @functools.partial(
        pl.kernel,
        mesh=mesh,
        compiler_params=pltpu.CompilerParams(needs_layout_passes=False),
        out_type=[
            jax.ShapeDtypeStruct((B,), jnp.float32),
            jax.ShapeDtypeStruct((B,), jnp.float32),
        ],
        scratch_types=[
            pltpu.VMEM((BPW,), jnp.int32),
            pltpu.VMEM((BPW,), jnp.int32),
            pltpu.VMEM((BPW,), jnp.int32),
            pltpu.VMEM((D, W), jnp.float32),
            pltpu.VMEM((D, W), jnp.float32),
            pltpu.VMEM((D, W), jnp.float32),
            pltpu.VMEM((D, W), jnp.float32),
            pltpu.VMEM((D, W), jnp.float32),
            pltpu.VMEM((D, W), jnp.float32),
            pltpu.VMEM((BPW,), jnp.float32),
            pltpu.VMEM((BPW,), jnp.float32),
            pltpu.SemaphoreType.DMA,
            pltpu.SemaphoreType.DMA,
        ],
    )
    def k(u_hbm, p_hbm, n_hbm, ut_hbm, it_hbm, ps_out, ns_out,
          uidx, pidx, nidx, wu0, wu1, wp0, wp1, wn0, wn1,
          ps_v, ns_v, sem0, sem1):
        wid = lax.axis_index("s") * NC + lax.axis_index("c")
        base = wid * BPW
        pltpu.sync_copy(u_hbm.at[pl.ds(base, BPW)], uidx)
        pltpu.sync_copy(p_hbm.at[pl.ds(base, BPW)], pidx)
        pltpu.sync_copy(n_hbm.at[pl.ds(base, BPW)], nidx)

        lane = lax.iota(jnp.int32, L)
        rows4 = [lax.iota(jnp.int32, L) + c * L for c in range(D // L)]

        def win_off(i):
            return i & jnp.int32(-W)

        bufs = ((wu0, wp0, wn0, sem0), (wu1, wp1, wn1, sem1))

        def group(g, carry):
            gbase = g * L
            uvec = uidx[pl.ds(gbase, L)]
            pvec = pidx[pl.ds(gbase, L)]
            nvec = nidx[pl.ds(gbase, L)]

            def issue(jj):
                bu, bp, bn, sem = bufs[jj % 2]
                iu, ip, iN = uvec[jj], pvec[jj], nvec[jj]
                ou, op, oN = win_off(iu), win_off(ip), win_off(iN)
                cs = (
                    pltpu.async_copy(
                        ut_hbm.at[:, pl.ds(pl.multiple_of(ou, W), W)], bu, sem),
                    pltpu.async_copy(
                        it_hbm.at[:, pl.ds(pl.multiple_of(op, W), W)], bp, sem),
                    pltpu.async_copy(
                        it_hbm.at[:, pl.ds(pl.multiple_of(oN, W), W)], bn, sem),
                )
                return cs, (iu - ou, ip - op, iN - oN)

            psel = jnp.zeros((L,), jnp.float32)
            nsel = jnp.zeros((L,), jnp.float32)
            cur = issue(0)
            for j in range(L):
                nxt = issue(j + 1) if j + 1 < L else None
                cs, (lu_s, lp_s, ln_s) = cur
                for c_ in cs:
                    c_.wait()
                bu, bp, bn, _ = bufs[j % 2]
                lu = jnp.full((L,), lu_s, jnp.int32)
                lp = jnp.full((L,), lp_s, jnp.int32)
                ln = jnp.full((L,), ln_s, jnp.int32)
                pacc = jnp.zeros((L,), jnp.float32)
                nacc = jnp.zeros((L,), jnp.float32)
                for c in range(D // L):
                    uv = plsc.load_gather(bu, [rows4[c], lu])
                    pv = plsc.load_gather(bp, [rows4[c], lp])
                    nv = plsc.load_gather(bn, [rows4[c], ln])
                    pacc = pacc + uv * pv
                    nacc = nacc + uv * nv
                mask = lane == j
                psel = jnp.where(mask, jnp.sum(pacc), psel)
                nsel = jnp.where(mask, jnp.sum(nacc), nsel)
                cur = nxt
            ps_v[pl.ds(gbase, L)] = psel
            ns_v[pl.ds(gbase, L)] = nsel
            return carry

        lax.fori_loop(0, NG, group, 0)
        pltpu.sync_copy(ps_v, ps_out.at[pl.ds(base, BPW)])
        pltpu.sync_copy(ns_v, ns_out.at[pl.ds(base, BPW)])

    return k(user, pos_item, neg_item, user_table_t, item_table_t)


def _tc_loss_sum(p, n):
    p2 = p.reshape(B, 1)
    n2 = n.reshape(1, B)

    def body(p_ref, n_ref, out_ref):
        i = pl.program_id(0)
        z = n_ref[...] - p_ref[...]
        m = jnp.maximum(z, 0.0)
        t = 1.0 + jnp.exp(-jnp.abs(z))
        acc = t[:, 0:128]
        for c in range(1, B // 128):
            acc = acc * t[:, c * 128:(c + 1) * 128]
        part = jnp.sum(m) + jnp.sum(jnp.log(acc))

        @pl.when(i == 0)
        def _():
            out_ref[...] = jnp.zeros_like(out_ref)

        out_ref[...] += jnp.reshape(part, (1, 1))

    out = pl.pallas_call(
        body,
        grid=(B // RB,),
        in_specs=[
            pl.BlockSpec((RB, 1), lambda i: (i, 0)),
            pl.BlockSpec((1, B), lambda i: (0, 0)),
        ],
        out_specs=pl.BlockSpec((1, 1), lambda i: (0, 0)),
        out_shape=jax.ShapeDtypeStruct((1, 1), jnp.float32),
    )(p2, n2)
    return out[0, 0]


def kernel(user, pos_item, neg_item, user_table, item_table):
    p, nvec = _sc_scores(user, pos_item, jnp.reshape(neg_item, (-1,)),
                         user_table.T, item_table.T)
    loss = _tc_loss_sum(p, nvec) / (B * B)
    logits = p.reshape(B, 1)
    return (logits, loss)


# speedup vs baseline: 4.8861x; 1.0282x over previous
import functools

import jax
import jax.numpy as jnp
from jax import lax
from jax.experimental import pallas as pl
from jax.experimental.pallas import tpu as pltpu
from jax.experimental.pallas import tpu_sc as plsc

B = 4096
D = 64
NROWS = 1000000
L = 16
NC = 2
NS = 16
NW = NC * NS
BPW = B // NW
NG = BPW // L
W = 128
WMAX = NROWS - W

RB = 512


def _sc_scores(user, pos_item, neg_item, user_table_t, item_table_t):
    mesh = plsc.VectorSubcoreMesh(core_axis_name="c", subcore_axis_name="s")

    @functools.partial(
        pl.kernel,
        mesh=mesh,
        compiler_params=pltpu.CompilerParams(needs_layout_passes=False),
        out_type=[
            jax.ShapeDtypeStruct((B,), jnp.float32),
            jax.ShapeDtypeStruct((B,), jnp.float32),
        ],
        scratch_types=[
            pltpu.VMEM((BPW,), jnp.int32),
            pltpu.VMEM((BPW,), jnp.int32),
            pltpu.VMEM((BPW,), jnp.int32),
            pltpu.VMEM((D, W), jnp.float32),
            pltpu.VMEM((D, W), jnp.float32),
            pltpu.VMEM((D, W), jnp.float32),
            pltpu.VMEM((D, W), jnp.float32),
            pltpu.VMEM((D, W), jnp.float32),
            pltpu.VMEM((D, W), jnp.float32),
            pltpu.VMEM((D, W), jnp.float32),
            pltpu.VMEM((D, W), jnp.float32),
            pltpu.VMEM((D, W), jnp.float32),
            pltpu.VMEM((BPW,), jnp.float32),
            pltpu.VMEM((BPW,), jnp.float32),
            pltpu.SemaphoreType.DMA,
            pltpu.SemaphoreType.DMA,
            pltpu.SemaphoreType.DMA,
        ],
    )
    def k(u_hbm, p_hbm, n_hbm, ut_hbm, it_hbm, ps_out, ns_out,
          uidx, pidx, nidx, wu0, wu1, wu2, wp0, wp1, wp2, wn0, wn1, wn2,
          ps_v, ns_v, sem0, sem1, sem2):
        wid = lax.axis_index("s") * NC + lax.axis_index("c")
        base = wid * BPW
        pltpu.sync_copy(u_hbm.at[pl.ds(base, BPW)], uidx)
        pltpu.sync_copy(p_hbm.at[pl.ds(base, BPW)], pidx)
        pltpu.sync_copy(n_hbm.at[pl.ds(base, BPW)], nidx)

        lane = lax.iota(jnp.int32, L)
        rows4 = [lax.iota(jnp.int32, L) + c * L for c in range(D // L)]

        def win_off(i):
            return i & jnp.int32(-W)

        bufs = ((wu0, wp0, wn0, sem0), (wu1, wp1, wn1, sem1),
                (wu2, wp2, wn2, sem2))
        NBUF = len(bufs)

        def group(g, carry):
            gbase = g * L
            uvec = uidx[pl.ds(gbase, L)]
            pvec = pidx[pl.ds(gbase, L)]
            nvec = nidx[pl.ds(gbase, L)]

            def issue(jj):
                bu, bp, bn, sem = bufs[jj % NBUF]
                iu, ip, iN = uvec[jj], pvec[jj], nvec[jj]
                ou, op, oN = win_off(iu), win_off(ip), win_off(iN)
                cs = (
                    pltpu.async_copy(
                        ut_hbm.at[:, pl.ds(pl.multiple_of(ou, W), W)], bu, sem),
                    pltpu.async_copy(
                        it_hbm.at[:, pl.ds(pl.multiple_of(op, W), W)], bp, sem),
                    pltpu.async_copy(
                        it_hbm.at[:, pl.ds(pl.multiple_of(oN, W), W)], bn, sem),
                )
                return cs, (iu - ou, ip - op, iN - oN)

            psel = jnp.zeros((L,), jnp.float32)
            nsel = jnp.zeros((L,), jnp.float32)
            inflight = [issue(0), issue(1)]
            for j in range(L):
                if j + 2 < L:
                    inflight.append(issue(j + 2))
                cs, (lu_s, lp_s, ln_s) = inflight.pop(0)
                for c_ in cs:
                    c_.wait()
                bu, bp, bn, _ = bufs[j % NBUF]
                lu = jnp.full((L,), lu_s, jnp.int32)
                lp = jnp.full((L,), lp_s, jnp.int32)
                ln = jnp.full((L,), ln_s, jnp.int32)
                pacc = jnp.zeros((L,), jnp.float32)
                nacc = jnp.zeros((L,), jnp.float32)
                for c in range(D // L):
                    uv = plsc.load_gather(bu, [rows4[c], lu])
                    pv = plsc.load_gather(bp, [rows4[c], lp])
                    nv = plsc.load_gather(bn, [rows4[c], ln])
                    pacc = pacc + uv * pv
                    nacc = nacc + uv * nv
                mask = lane == j
                psel = jnp.where(mask, jnp.sum(pacc), psel)
                nsel = jnp.where(mask, jnp.sum(nacc), nsel)
            ps_v[pl.ds(gbase, L)] = psel
            ns_v[pl.ds(gbase, L)] = nsel
            return carry

        lax.fori_loop(0, NG, group, 0)
        pltpu.sync_copy(ps_v, ps_out.at[pl.ds(base, BPW)])
        pltpu.sync_copy(ns_v, ns_out.at[pl.ds(base, BPW)])

    return k(user, pos_item, neg_item, user_table_t, item_table_t)


def _tc_loss_sum(p, n):
    p2 = p.reshape(B, 1)
    n2 = n.reshape(1, B)

    def body(p_ref, n_ref, out_ref):
        i = pl.program_id(0)
        z = n_ref[...] - p_ref[...]
        m = jnp.maximum(z, 0.0)
        t = 1.0 + jnp.exp(-jnp.abs(z))
        acc = t[:, 0:128]
        for c in range(1, B // 128):
            acc = acc * t[:, c * 128:(c + 1) * 128]
        part = jnp.sum(m) + jnp.sum(jnp.log(acc))

        @pl.when(i == 0)
        def _():
            out_ref[...] = jnp.zeros_like(out_ref)

        out_ref[...] += jnp.reshape(part, (1, 1))

    out = pl.pallas_call(
        body,
        grid=(B // RB,),
        in_specs=[
            pl.BlockSpec((RB, 1), lambda i: (i, 0)),
            pl.BlockSpec((1, B), lambda i: (0, 0)),
        ],
        out_specs=pl.BlockSpec((1, 1), lambda i: (0, 0)),
        out_shape=jax.ShapeDtypeStruct((1, 1), jnp.float32),
    )(p2, n2)
    return out[0, 0]


def kernel(user, pos_item, neg_item, user_table, item_table):
    p, nvec = _sc_scores(user, pos_item, jnp.reshape(neg_item, (-1,)),
                         user_table.T, item_table.T)
    loss = _tc_loss_sum(p, nvec) / (B * B)
    logits = p.reshape(B, 1)
    return (logits, loss)
